# baseline (speedup 1.0000x reference)
import functools

import jax
import jax.numpy as jnp
from jax import lax
from jax.experimental import pallas as pl
from jax.experimental.pallas import tpu as pltpu
from jax.experimental.pallas import tpu_sc as plsc

_TARGET = 0

N_ROWS = 1_000_000
ROW = 8
L = 16
NC, NS = 2, 16
NW = NC * NS

GROUP_WORDS = L * ROW
TOTAL_GROUPS = N_ROWS // L
GROUPS_PER_W = TOTAL_GROUPS // NW
TAIL_GROUPS = TOTAL_GROUPS - GROUPS_PER_W * NW
CHUNK_GROUPS = 217
NCHUNKS = GROUPS_PER_W // CHUNK_GROUPS
CHUNK_WORDS = CHUNK_GROUPS * GROUP_WORDS


def _sc_partials_body(x_hbm, out_hbm, buf0, buf1, tailbuf, acc_ref, sem0, sem1):
    cid = lax.axis_index("c")
    sid = lax.axis_index("s")
    wid = sid * NC + cid

    base_word = pl.multiple_of(wid * (GROUPS_PER_W * GROUP_WORDS), GROUP_WORDS)
    acc_ref[...] = jnp.zeros((L,), jnp.float32)
    iota8 = lax.iota(jnp.int32, 16) * ROW

    bufs = (buf0, buf1)
    sems = (sem0, sem1)

    def start(c):
        src = x_hbm.at[pl.ds(base_word + c * CHUNK_WORDS, CHUNK_WORDS)]
        return pltpu.async_copy(src, bufs[c % 2], sems[c % 2])

    cps = [None, None]
    cps[0] = start(0)
    for c in range(NCHUNKS):
        if c + 1 < NCHUNKS:
            cps[(c + 1) % 2] = start(c + 1)
        cps[c % 2].wait()
        buf = bufs[c % 2]

        def group_body(g, acc, buf=buf):
            base = g * GROUP_WORDS
            cls = plsc.load_gather(buf, [base + iota8])
            sc = plsc.load_gather(buf, [base + iota8 + (ROW - 1)])
            keep = cls.astype(jnp.int32) == _TARGET
            return acc + jnp.where(keep, sc, 0.0)

        acc_ref[...] = lax.fori_loop(0, CHUNK_GROUPS, group_body, acc_ref[...])

    @pl.when(wid < TAIL_GROUPS)
    def _():
        off = pl.multiple_of(
            (NW * GROUPS_PER_W + wid) * GROUP_WORDS, GROUP_WORDS
        )
        pltpu.sync_copy(x_hbm.at[pl.ds(off, GROUP_WORDS)], tailbuf)
        cls = plsc.load_gather(tailbuf, [iota8])
        sc = plsc.load_gather(tailbuf, [iota8 + (ROW - 1)])
        keep = cls.astype(jnp.int32) == _TARGET
        acc_ref[...] = acc_ref[...] + jnp.where(keep, sc, 0.0)

    pltpu.sync_copy(acc_ref, out_hbm.at[wid])


_sc_partials = pl.kernel(
    _sc_partials_body,
    out_type=jax.ShapeDtypeStruct((NW, L), jnp.float32),
    mesh=plsc.VectorSubcoreMesh(
        core_axis_name="c", subcore_axis_name="s", num_cores=NC, num_subcores=NS
    ),
    compiler_params=pltpu.CompilerParams(needs_layout_passes=False),
    scratch_types=[
        pltpu.VMEM((CHUNK_WORDS,), jnp.float32),
        pltpu.VMEM((CHUNK_WORDS,), jnp.float32),
        pltpu.VMEM((GROUP_WORDS,), jnp.float32),
        pltpu.VMEM((L,), jnp.float32),
        pltpu.SemaphoreType.DMA,
        pltpu.SemaphoreType.DMA,
    ],
)


def _finish_body(p_ref, o_ref):
    o_ref[0, 0] = -jnp.sum(p_ref[...])


_finish = pl.pallas_call(
    _finish_body,
    out_shape=jax.ShapeDtypeStruct((1, 1), jnp.float32),
    out_specs=pl.BlockSpec(memory_space=pltpu.SMEM),
)


@jax.jit
def kernel(box3d_branch):
    flat = box3d_branch.reshape(N_ROWS * ROW)
    partials = _sc_partials(flat)
    return _finish(partials)[0, 0]


# speedup vs baseline: 1.0023x; 1.0023x over previous
import functools

import jax
import jax.numpy as jnp
from jax import lax
from jax.experimental import pallas as pl
from jax.experimental.pallas import tpu as pltpu
from jax.experimental.pallas import tpu_sc as plsc

_TARGET = 0

N_ROWS = 1_000_000
ROW = 8
L = 16
NC, NS = 2, 16
NW = NC * NS

TOTAL_GROUPS = N_ROWS // L
GROUPS_PER_W = TOTAL_GROUPS // NW
TAIL_GROUPS = TOTAL_GROUPS - GROUPS_PER_W * NW
CHUNK_GROUPS = 217
NCHUNKS = GROUPS_PER_W // CHUNK_GROUPS
CHUNK_ROWS = CHUNK_GROUPS * L


def _sc_partials_body(x_hbm, out_hbm, buf0, buf1, tailbuf, acc_ref, sem0, sem1):
    cid = lax.axis_index("c")
    sid = lax.axis_index("s")
    wid = sid * NC + cid

    base_row = pl.multiple_of(wid * (GROUPS_PER_W * L), L)
    acc_ref[...] = jnp.zeros((L,), jnp.float32)
    iota = lax.iota(jnp.int32, 16)
    col0 = jnp.zeros((16,), jnp.int32)
    col7 = jnp.full((16,), ROW - 1, jnp.int32)

    bufs = (buf0, buf1)
    sems = (sem0, sem1)

    def start(c):
        src = x_hbm.at[pl.ds(base_row + c * CHUNK_ROWS, CHUNK_ROWS), :]
        return pltpu.async_copy(src, bufs[c % 2], sems[c % 2])

    cps = [None, None]
    cps[0] = start(0)
    for c in range(NCHUNKS):
        if c + 1 < NCHUNKS:
            cps[(c + 1) % 2] = start(c + 1)
        cps[c % 2].wait()
        buf = bufs[c % 2]

        def group_body(g, acc, buf=buf):
            rows = g * L + iota
            cls = plsc.load_gather(buf, [rows, col0])
            sc = plsc.load_gather(buf, [rows, col7])
            keep = cls.astype(jnp.int32) == _TARGET
            return acc + jnp.where(keep, sc, 0.0)

        acc_ref[...] = lax.fori_loop(0, CHUNK_GROUPS, group_body, acc_ref[...])

    @pl.when(wid < TAIL_GROUPS)
    def _():
        off = pl.multiple_of((NW * GROUPS_PER_W + wid) * L, L)
        pltpu.sync_copy(x_hbm.at[pl.ds(off, L), :], tailbuf)
        cls = plsc.load_gather(tailbuf, [iota, col0])
        sc = plsc.load_gather(tailbuf, [iota, col7])
        keep = cls.astype(jnp.int32) == _TARGET
        acc_ref[...] = acc_ref[...] + jnp.where(keep, sc, 0.0)

    pltpu.sync_copy(acc_ref, out_hbm.at[wid])


_sc_partials = pl.kernel(
    _sc_partials_body,
    out_type=jax.ShapeDtypeStruct((NW, L), jnp.float32),
    mesh=plsc.VectorSubcoreMesh(
        core_axis_name="c", subcore_axis_name="s", num_cores=NC, num_subcores=NS
    ),
    compiler_params=pltpu.CompilerParams(
        needs_layout_passes=False, use_tc_tiling_on_sc=False
    ),
    scratch_types=[
        pltpu.VMEM((CHUNK_ROWS, ROW), jnp.float32),
        pltpu.VMEM((CHUNK_ROWS, ROW), jnp.float32),
        pltpu.VMEM((L, ROW), jnp.float32),
        pltpu.VMEM((L,), jnp.float32),
        pltpu.SemaphoreType.DMA,
        pltpu.SemaphoreType.DMA,
    ],
)


def _finish_body(p_ref, o_ref):
    o_ref[0, 0] = -jnp.sum(p_ref[...])


_finish = pl.pallas_call(
    _finish_body,
    out_shape=jax.ShapeDtypeStruct((1, 1), jnp.float32),
    out_specs=pl.BlockSpec(memory_space=pltpu.SMEM),
)


@jax.jit
def kernel(box3d_branch):
    partials = _sc_partials(box3d_branch)
    return _finish(partials)[0, 0]


# speedup vs baseline: 13.4873x; 13.4565x over previous
import functools

import jax
import jax.numpy as jnp
from jax import lax
from jax.experimental import pallas as pl
from jax.experimental.pallas import tpu as pltpu
from jax.experimental.pallas import tpu_sc as plsc

_TARGET = 0

N_ROWS = 1_000_000
ROW = 8
L = 16
NC, NS = 2, 16
NW = NC * NS
LANE = 128

FULL_TILES = N_ROWS // LANE
REM = N_ROWS - FULL_TILES * LANE
TILES_PER_W = FULL_TILES // NW
EXTRA_TILES = FULL_TILES - TILES_PER_W * NW
CHUNK_TILES = 61
NCHUNKS = TILES_PER_W // CHUNK_TILES
CHUNK_LANES = CHUNK_TILES * LANE
GROUPS_PER_CHUNK = CHUNK_LANES // L
WORDS_PER_W = TILES_PER_W * LANE

_CLS, _SCORE = 0, ROW - 1


def _sc_partials_body(xt_hbm, out_hbm, buf0, buf1, tbuf, acc_ref, sem0, sem1):
    cid = lax.axis_index("c")
    sid = lax.axis_index("s")
    wid = sid * NC + cid

    base = pl.multiple_of(wid * WORDS_PER_W, LANE)

    bufs = (buf0, buf1)
    sems = (sem0, sem1)

    def start(c):
        src = xt_hbm.at[:, pl.ds(base + c * CHUNK_LANES, CHUNK_LANES)]
        return pltpu.async_copy(src, bufs[c % 2], sems[c % 2])

    def make_group_body(buf):
        def group_body(g, acc):
            cls = buf[_CLS, pl.ds(g * L, L)]
            sc = buf[_SCORE, pl.ds(g * L, L)]
            keep = cls.astype(jnp.int32) == _TARGET
            return acc + jnp.where(keep, sc, 0.0)

        return group_body

    acc = jnp.zeros((L,), jnp.float32)
    cps = [None, None]
    cps[0] = start(0)
    for c in range(NCHUNKS):
        if c + 1 < NCHUNKS:
            cps[(c + 1) % 2] = start(c + 1)
        cps[c % 2].wait()
        acc = plsc.parallel_loop(0, GROUPS_PER_CHUNK, unroll=8, carry=acc)(
            make_group_body(bufs[c % 2])
        )
    acc_ref[...] = acc

    @pl.when(wid < EXTRA_TILES)
    def _():
        off = pl.multiple_of((NW * TILES_PER_W + wid) * LANE, LANE)
        pltpu.sync_copy(xt_hbm.at[:, pl.ds(off, LANE)], tbuf)
        acc_ref[...] = lax.fori_loop(
            0, LANE // L, make_group_body(tbuf), acc_ref[...]
        )

    pltpu.sync_copy(acc_ref, out_hbm.at[wid])


_sc_partials = pl.kernel(
    _sc_partials_body,
    out_type=jax.ShapeDtypeStruct((NW, L), jnp.float32),
    mesh=plsc.VectorSubcoreMesh(
        core_axis_name="c", subcore_axis_name="s", num_cores=NC, num_subcores=NS
    ),
    compiler_params=pltpu.CompilerParams(
        needs_layout_passes=False, use_tc_tiling_on_sc=True
    ),
    scratch_types=[
        pltpu.VMEM((ROW, CHUNK_LANES), jnp.float32),
        pltpu.VMEM((ROW, CHUNK_LANES), jnp.float32),
        pltpu.VMEM((ROW, LANE), jnp.float32),
        pltpu.VMEM((L,), jnp.float32),
        pltpu.SemaphoreType.DMA,
        pltpu.SemaphoreType.DMA,
    ],
)


def _finish_body(p_ref, x_ref, o_ref):
    cls = x_ref[_CLS : _CLS + 1, :]
    sc = x_ref[_SCORE : _SCORE + 1, :]
    valid = lax.broadcasted_iota(jnp.int32, (1, LANE), 1) < REM
    keep = jnp.logical_and(cls.astype(jnp.int32) == _TARGET, valid)
    tail = jnp.sum(jnp.where(keep, sc, 0.0))
    o_ref[0, 0] = -(jnp.sum(p_ref[...]) + tail)


_finish = pl.pallas_call(
    _finish_body,
    out_shape=jax.ShapeDtypeStruct((1, 1), jnp.float32),
    grid=(1,),
    in_specs=[
        pl.BlockSpec((NW, L), lambda i: (0, 0)),
        pl.BlockSpec((ROW, LANE), lambda i: (0, FULL_TILES)),
    ],
    out_specs=pl.BlockSpec((1, 1), lambda i: (0, 0), memory_space=pltpu.SMEM),
)


@jax.jit
def kernel(box3d_branch):
    xt = box3d_branch.T
    partials = _sc_partials(xt)
    return _finish(partials, xt)[0, 0]
